# Initial kernel scaffold; baseline (speedup 1.0000x reference)
import jax
import jax.numpy as jnp
from jax.experimental import pallas as pl


def kernel(emb_ent, emb_rel, triplets, attn_W, attn_b, attn_vec, aggr_W, aggr_b):
    raise NotImplementedError("write your pallas kernel here")


# speedup vs baseline: 18.3483x; 18.3483x over previous
import jax
import jax.numpy as jnp
from jax import lax
from jax.experimental import pallas as pl
from jax.experimental.pallas import tpu as pltpu
from jax.experimental.pallas import tpu_sc as plsc

N_ENT = 10000
N_REL = 10000
NT = 320000
DIN = 128
DR = 16
H = 8
DH = 16

NC = 2
NS = 16
NW = NC * NS
EPW = NT // NW
CH = 80
NCHUNK = EPW // CH
N_PAD = 10240
RPT = N_PAD // NS
ZCH = 128

_mesh = plsc.VectorSubcoreMesh(core_axis_name="c", subcore_axis_name="s")


def _zero_vmem(buf, nrow, ncol):
    zv = jnp.zeros((16,), jnp.float32)

    def body(i, _):
        for k in range(ncol // 16):
            buf[i, pl.ds(16 * k, 16)] = zv
        return 0

    lax.fori_loop(0, nrow, body, 0)


def _k1_body(aug_hbm, rel_hbm, tail_hbm, out_hbm, acc, idx_r, idx_t, rows, zbuf):
    c = lax.axis_index("c")
    s = lax.axis_index("s")
    wid = s * NC + c

    _zero_vmem(zbuf, ZCH, DIN)
    for q in range(RPT // ZCH):
        pltpu.sync_copy(zbuf, acc.at[pl.ds(s * RPT + q * ZCH, ZCH)])
    plsc.subcore_barrier()

    def chunk(ci, _):
        off = wid * EPW + ci * CH
        pltpu.sync_copy(rel_hbm.at[pl.ds(off, CH)], idx_r)
        pltpu.sync_copy(tail_hbm.at[pl.ds(off, CH)], idx_t)
        pltpu.sync_copy(aug_hbm.at[idx_r], rows)
        pltpu.sync_copy(rows, acc.at[idx_t], add=True)
        return 0

    lax.fori_loop(0, NCHUNK, chunk, 0)
    plsc.subcore_barrier()
    for q in range(RPT // ZCH):
        r0 = s * RPT + q * ZCH
        pltpu.sync_copy(acc.at[pl.ds(r0, ZCH)], out_hbm.at[c, pl.ds(r0, ZCH)])


_k1 = pl.kernel(
    _k1_body,
    out_type=jax.ShapeDtypeStruct((NC, N_PAD, DIN), jnp.float32),
    mesh=_mesh,
    scratch_types=[
        pltpu.VMEM_SHARED((N_PAD, DIN), jnp.float32),
        pltpu.VMEM((CH,), jnp.int32),
        pltpu.VMEM((CH,), jnp.int32),
        pltpu.VMEM((CH, DIN), jnp.float32),
        pltpu.VMEM((ZCH, DIN), jnp.float32),
    ],
)


def _k3_body(tt_hbm, hh_hbm, rw_hbm, head_hbm, rel_hbm, tail_hbm, z_hbm,
             idx_h, idx_r, idx_t, bt, bh, br):
    c = lax.axis_index("c")
    s = lax.axis_index("s")
    wid = s * NC + c

    def chunk(ci, _):
        off = wid * EPW + ci * CH
        pltpu.sync_copy(head_hbm.at[pl.ds(off, CH)], idx_h)
        pltpu.sync_copy(rel_hbm.at[pl.ds(off, CH)], idx_r)
        pltpu.sync_copy(tail_hbm.at[pl.ds(off, CH)], idx_t)
        pltpu.sync_copy(tt_hbm.at[idx_t], bt)
        pltpu.sync_copy(hh_hbm.at[idx_h], bh)
        pltpu.sync_copy(rw_hbm.at[idx_r], br)

        def row(e, _):
            for k in range(DIN // 16):
                sl = pl.ds(16 * k, 16)
                bt[e, sl] = bt[e, sl] + bh[e, sl] + br[e, sl]
            return 0

        lax.fori_loop(0, CH, row, 0)
        pltpu.sync_copy(bt, z_hbm.at[pl.ds(off, CH)])
        return 0

    lax.fori_loop(0, NCHUNK, chunk, 0)


_k3 = pl.kernel(
    _k3_body,
    out_type=jax.ShapeDtypeStruct((NT, DIN), jnp.float32),
    mesh=_mesh,
    scratch_types=[
        pltpu.VMEM((CH,), jnp.int32),
        pltpu.VMEM((CH,), jnp.int32),
        pltpu.VMEM((CH,), jnp.int32),
        pltpu.VMEM((CH, DIN), jnp.float32),
        pltpu.VMEM((CH, DIN), jnp.float32),
        pltpu.VMEM((CH, DIN), jnp.float32),
    ],
)


def _k7_body(aw_hbm, ar_hbm, e_hbm, head_hbm, rel_hbm, tail_hbm,
             o_hbm, ssum_hbm,
             acc_o, idx_h, idx_r, idx_t, ba, bb, be, zbuf):
    c = lax.axis_index("c")
    s = lax.axis_index("s")
    wid = s * NC + c

    _zero_vmem(zbuf, ZCH, DIN)
    for q in range(RPT // ZCH):
        r0 = s * RPT + q * ZCH
        pltpu.sync_copy(zbuf, acc_o.at[pl.ds(r0, ZCH)])
    plsc.subcore_barrier()

    def chunk(ci, _):
        off = wid * EPW + ci * CH
        pltpu.sync_copy(head_hbm.at[pl.ds(off, CH)], idx_h)
        pltpu.sync_copy(rel_hbm.at[pl.ds(off, CH)], idx_r)
        pltpu.sync_copy(tail_hbm.at[pl.ds(off, CH)], idx_t)
        pltpu.sync_copy(e_hbm.at[pl.ds(off, CH)], be)
        pltpu.sync_copy(aw_hbm.at[idx_h], ba)
        pltpu.sync_copy(ar_hbm.at[idx_r], bb)

        def row(e, _):
            ev = be[e, pl.ds(0, 16)]
            for j in range(H):
                sl = pl.ds(16 * j, 16)
                ba[e, sl] = (ba[e, sl] + bb[e, sl]) * ev[j]
            return 0

        lax.fori_loop(0, CH, row, 0)
        pltpu.sync_copy(ba, acc_o.at[idx_t], add=True)
        return 0

    lax.fori_loop(0, NCHUNK, chunk, 0)
    plsc.subcore_barrier()
    for q in range(RPT // ZCH):
        r0 = s * RPT + q * ZCH
        pltpu.sync_copy(acc_o.at[pl.ds(r0, ZCH)], o_hbm.at[c, pl.ds(r0, ZCH)])
    plsc.subcore_barrier()

    _zero_vmem(ba, CH, DIN)
    for q in range(RPT // ZCH):
        r0 = s * RPT + q * ZCH
        pltpu.sync_copy(zbuf, acc_o.at[pl.ds(r0, ZCH)])
    plsc.subcore_barrier()

    def chunk_b(ci, _):
        off = wid * EPW + ci * CH
        pltpu.sync_copy(tail_hbm.at[pl.ds(off, CH)], idx_t)
        pltpu.sync_copy(e_hbm.at[pl.ds(off, CH)], be)

        def row(e, _):
            ba[e, pl.ds(0, 16)] = be[e, pl.ds(0, 16)]
            return 0

        lax.fori_loop(0, CH, row, 0)
        pltpu.sync_copy(ba, acc_o.at[idx_t], add=True)
        return 0

    lax.fori_loop(0, NCHUNK, chunk_b, 0)
    plsc.subcore_barrier()
    for q in range(RPT // ZCH):
        r0 = s * RPT + q * ZCH
        pltpu.sync_copy(acc_o.at[pl.ds(r0, ZCH)], ssum_hbm.at[c, pl.ds(r0, ZCH)])


_k7 = pl.kernel(
    _k7_body,
    out_type=[
        jax.ShapeDtypeStruct((NC, N_PAD, DIN), jnp.float32),
        jax.ShapeDtypeStruct((NC, N_PAD, DIN), jnp.float32),
    ],
    mesh=_mesh,
    scratch_types=[
        pltpu.VMEM_SHARED((N_PAD, DIN), jnp.float32),
        pltpu.VMEM((CH,), jnp.int32),
        pltpu.VMEM((CH,), jnp.int32),
        pltpu.VMEM((CH,), jnp.int32),
        pltpu.VMEM((CH, DIN), jnp.float32),
        pltpu.VMEM((CH, DIN), jnp.float32),
        pltpu.VMEM((CH, 16), jnp.float32),
        pltpu.VMEM((ZCH, DIN), jnp.float32),
    ],
)


def _attn_blockdiag(avf):
    di = lax.broadcasted_iota(jnp.int32, (DIN, 16), 0) // DH
    ji = lax.broadcasted_iota(jnp.int32, (DIN, 16), 1)
    m = jnp.where(di == ji, 1.0, 0.0)
    c16 = jnp.dot(jnp.abs(avf)[None, :], m,
                  preferred_element_type=jnp.float32)[0]
    return avf, m, c16


_K2B = 1000


def _k2_body(emb_ref, rel_ref, sp_ref, aw_ref, ab_ref, av_ref, gw_ref, gb_ref,
             tt_o, hh_o, rw_o, ag_o, ar_o, es_o, as_o):
    emb = emb_ref[...]
    rel = rel_ref[...]
    W = aw_ref[...]
    Wt, Wh, Wr = W[:, :DIN], W[:, DIN:2 * DIN], W[:, 2 * DIN:]
    GW = gw_ref[...]
    AhW, ArW = GW[:, :DIN], GW[:, DIN:]

    S = sp_ref[0] + sp_ref[1]
    self_rel = S[:, :DR] / (S[:, DR:DR + 1] + 1e-16)

    tt = jnp.dot(emb, Wt.T, preferred_element_type=jnp.float32) + ab_ref[...]
    hh = jnp.dot(emb, Wh.T, preferred_element_type=jnp.float32)
    rw = jnp.dot(rel, Wr.T, preferred_element_type=jnp.float32)
    ag = jnp.dot(emb, AhW.T, preferred_element_type=jnp.float32) + gb_ref[...]
    ar = jnp.dot(rel, ArW.T, preferred_element_type=jnp.float32)

    avf, m, c16 = _attn_blockdiag(av_ref[...])
    zs = tt + hh + jnp.dot(self_rel, Wr.T, preferred_element_type=jnp.float32)
    es = jnp.exp(jnp.dot(jnp.tanh(zs) * avf, m, preferred_element_type=jnp.float32) - c16)
    a_s = ag + jnp.dot(self_rel, ArW.T, preferred_element_type=jnp.float32)

    tt_o[...] = tt
    hh_o[...] = hh
    rw_o[...] = rw
    ag_o[...] = ag
    ar_o[...] = ar
    es_o[...] = es
    as_o[...] = a_s


def _k2(emb_ent, emb_rel, sparts, attn_W, attn_b, attn_vec, aggr_W, aggr_b):
    n = N_ENT // _K2B
    full = lambda shape: pl.BlockSpec(shape, lambda i: tuple(0 for _ in shape))
    row = lambda w: pl.BlockSpec((_K2B, w), lambda i: (i, 0))
    return pl.pallas_call(
        _k2_body,
        grid=(n,),
        in_specs=[
            row(DIN),
            row(DR),
            pl.BlockSpec((NC, _K2B, DIN), lambda i: (0, i, 0)),
            full((DIN, 2 * DIN + DR)),
            full((DIN,)),
            full((DIN,)),
            full((DIN, DIN + DR)),
            full((DIN,)),
        ],
        out_specs=[row(DIN), row(DIN), row(DIN), row(DIN), row(DIN),
                   row(16), row(DIN)],
        out_shape=[
            jax.ShapeDtypeStruct((N_ENT, DIN), jnp.float32),
            jax.ShapeDtypeStruct((N_ENT, DIN), jnp.float32),
            jax.ShapeDtypeStruct((N_REL, DIN), jnp.float32),
            jax.ShapeDtypeStruct((N_ENT, DIN), jnp.float32),
            jax.ShapeDtypeStruct((N_REL, DIN), jnp.float32),
            jax.ShapeDtypeStruct((N_ENT, 16), jnp.float32),
            jax.ShapeDtypeStruct((N_ENT, DIN), jnp.float32),
        ],
    )(emb_ent, emb_rel, sparts, attn_W, attn_b, attn_vec, aggr_W, aggr_b)


_K4B = 2000


def _k4_body(z_ref, av_ref, e_o):
    avf, m, c16 = _attn_blockdiag(av_ref[...])
    u = jnp.tanh(z_ref[...])
    e_o[...] = jnp.exp(jnp.dot(u * avf, m, preferred_element_type=jnp.float32) - c16)


def _k4(z, attn_vec):
    n = NT // _K4B
    return pl.pallas_call(
        _k4_body,
        grid=(n,),
        in_specs=[
            pl.BlockSpec((_K4B, DIN), lambda i: (i, 0)),
            pl.BlockSpec((DIN,), lambda i: (0,)),
        ],
        out_specs=pl.BlockSpec((_K4B, 16), lambda i: (i, 0)),
        out_shape=jax.ShapeDtypeStruct((NT, 16), jnp.float32),
    )(z, attn_vec)


_K8B = 1000


def _k8_body(op_ref, sp_ref, es_ref, as_ref, out_o):
    O = op_ref[0] + op_ref[1]
    Ssum = sp_ref[0][:, :H] + sp_ref[1][:, :H]
    es = es_ref[...][:, :H]
    a_s = as_ref[...].reshape(_K8B, H, DH)
    num = O.reshape(_K8B, H, DH) + es[:, :, None] * a_s
    den = (Ssum + es + 1e-16)[:, :, None]
    out_o[...] = (num / den).reshape(_K8B, DIN)


def _k8(oparts, sparts, es, a_s):
    n = N_ENT // _K8B
    return pl.pallas_call(
        _k8_body,
        grid=(n,),
        in_specs=[
            pl.BlockSpec((NC, _K8B, DIN), lambda i: (0, i, 0)),
            pl.BlockSpec((NC, _K8B, DIN), lambda i: (0, i, 0)),
            pl.BlockSpec((_K8B, 16), lambda i: (i, 0)),
            pl.BlockSpec((_K8B, DIN), lambda i: (i, 0)),
        ],
        out_specs=pl.BlockSpec((_K8B, DIN), lambda i: (i, 0)),
        out_shape=jax.ShapeDtypeStruct((N_ENT, DIN), jnp.float32),
    )(oparts, sparts, es, a_s)


def kernel(emb_ent, emb_rel, triplets, attn_W, attn_b, attn_vec, aggr_W, aggr_b):
    h_i = triplets[:, 0]
    r_i = triplets[:, 1]
    t_i = triplets[:, 2]

    aug = jnp.concatenate(
        [emb_rel, jnp.ones((N_REL, 1), jnp.float32),
         jnp.zeros((N_REL, DIN - DR - 1), jnp.float32)], axis=1)

    avf = attn_vec.reshape(DIN)

    sparts = _k1(aug, r_i, t_i)
    tt, hh, rw, ag, ar, es, a_s = _k2(
        emb_ent, emb_rel, sparts, attn_W, attn_b, avf, aggr_W, aggr_b)
    z = _k3(tt, hh, rw, h_i, r_i, t_i)
    e = _k4(z, avf)
    oparts, s2parts = _k7(ag, ar, e, h_i, r_i, t_i)
    return _k8(oparts, s2parts, es, a_s)
